# baseline (speedup 1.0000x reference)
import functools

import jax
import jax.numpy as jnp
from jax import lax
from jax.experimental import pallas as pl
from jax.experimental.pallas import tpu as pltpu
from jax.experimental.pallas import tpu_sc as plsc

N = 10000
E = 320000
D_IN = 128
DH = 64
NEG_SLOPE = 0.01

NC = 2
NS = 16
NW = NC * NS
CH = 128
NCHUNK = 80
E_PAD = NW * NCHUNK * CH
NP = 10112
ROWS_PER_TILE = NP // NS
BR = NP // 4
NBUF = 2
PF = 1

_MESH_CACHE = []


def _mesh():
    if not _MESH_CACHE:
        _MESH_CACHE.append(
            plsc.VectorSubcoreMesh(core_axis_name="c", subcore_axis_name="s",
                                   num_cores=NC, num_subcores=NS))
    return _MESH_CACHE[0]


def _zero_fill(buf, nrows, width):
    zv = jnp.zeros((16,), jnp.float32)

    def body(t, _):
        i = t // (width // 16)
        k = t % (width // 16)
        buf[i, pl.ds(k * 16, 16)] = zv
        return 0

    lax.fori_loop(0, nrows * (width // 16), body, 0)


def _ones_fill(buf, nrows, width):
    ov = jnp.ones((16,), jnp.float32)

    def body(t, _):
        i = t // (width // 16)
        k = t % (width // 16)
        buf[i, pl.ds(k * 16, 16)] = ov
        return 0

    lax.fori_loop(0, nrows * (width // 16), body, 0)


def _zero_stripe(acc_sp, z_v, base, width):
    nfull = ROWS_PER_TILE // CH
    rem = ROWS_PER_TILE - nfull * CH

    def body(k, _):
        pltpu.sync_copy(z_v, acc_sp.at[pl.ds(base + k * CH, CH)])
        return 0

    lax.fori_loop(0, nfull, body, 0)
    if rem:
        pltpu.sync_copy(z_v.at[pl.ds(0, rem)],
                        acc_sp.at[pl.ds(base + nfull * CH, rem)])


def _sc_degree_body(dst_hbm, out_hbm, idx_v, ones_v, z_v, deg_sp):
    cid = lax.axis_index("c")
    sid = lax.axis_index("s")
    wid = sid * NC + cid
    base = sid * ROWS_PER_TILE

    _ones_fill(ones_v, CH, 16)
    _zero_fill(z_v, CH, 16)
    _zero_stripe(deg_sp, z_v, base, 16)
    pltpu.sync_copy(dst_hbm.at[wid], idx_v)
    plsc.subcore_barrier()

    def body(j, _):
        pltpu.sync_copy(ones_v, deg_sp.at[idx_v.at[j]], add=True)
        return 0

    lax.fori_loop(0, NCHUNK, body, 0)
    plsc.subcore_barrier()
    pltpu.sync_copy(deg_sp.at[pl.ds(base, ROWS_PER_TILE)],
                    out_hbm.at[cid, pl.ds(base, ROWS_PER_TILE)])


def _sc_scatter_body(y_hbm, src_hbm, dst_hbm, out_hbm,
                     src_v, dst_v, rows_v, z_v, acc_sp, y_sp,
                     g0, g1, s0, s1):
    cid = lax.axis_index("c")
    sid = lax.axis_index("s")
    wid = sid * NC + cid
    base = sid * ROWS_PER_TILE
    gsem = (g0, g1)
    ssem = (s0, s1)

    pltpu.async_copy(y_hbm.at[pl.ds(base, ROWS_PER_TILE)],
                     y_sp.at[pl.ds(base, ROWS_PER_TILE)], g0)
    _zero_fill(z_v, CH, DH)
    _zero_stripe(acc_sp, z_v, base, DH)
    pltpu.sync_copy(src_hbm.at[wid], src_v)
    pltpu.sync_copy(dst_hbm.at[wid], dst_v)
    pltpu.make_async_copy(y_hbm.at[pl.ds(base, ROWS_PER_TILE)],
                          y_sp.at[pl.ds(base, ROWS_PER_TILE)], g0).wait()
    plsc.subcore_barrier()

    for u in range(PF):
        pltpu.async_copy(y_sp.at[src_v.at[u]], rows_v.at[u], gsem[u])

    def macro(m, _):
        j0 = m * NBUF

        for u in range(NBUF):
            j = j0 + u
            bg = (u + PF) % NBUF

            pltpu.make_async_copy(y_sp.at[src_v.at[j]], rows_v.at[u],
                                  gsem[u]).wait()

            @pl.when(j + PF < NCHUNK)
            def _(j=j, u=u, bg=bg):
                pltpu.async_copy(y_sp.at[src_v.at[j + PF]], rows_v.at[bg],
                                 gsem[bg])

            pltpu.sync_copy(rows_v.at[u], acc_sp.at[dst_v.at[j]], add=True)

        return 0

    lax.fori_loop(0, NCHUNK // NBUF, macro, 0)
    plsc.subcore_barrier()
    pltpu.sync_copy(acc_sp.at[pl.ds(base, ROWS_PER_TILE)],
                    out_hbm.at[cid, pl.ds(base, ROWS_PER_TILE)])


_SC_KERNELS = {}


def _sc_degree(dst_r):
    if "deg" not in _SC_KERNELS:
        _SC_KERNELS["deg"] = pl.kernel(
            _sc_degree_body,
            out_type=jax.ShapeDtypeStruct((NC, NP, 16), jnp.float32),
            mesh=_mesh(),
            scratch_types=[
                pltpu.VMEM((NCHUNK, CH), jnp.int32),
                pltpu.VMEM((CH, 16), jnp.float32),
                pltpu.VMEM((CH, 16), jnp.float32),
                pltpu.VMEM_SHARED((NP, 16), jnp.float32),
            ],
            compiler_params=pltpu.CompilerParams(use_tc_tiling_on_sc=False),
        )
    return _SC_KERNELS["deg"](dst_r)


def _sc_scatter(y, src_r, dst_r):
    if "scat" not in _SC_KERNELS:
        _SC_KERNELS["scat"] = pl.kernel(
            _sc_scatter_body,
            out_type=jax.ShapeDtypeStruct((NC, NP, DH), jnp.float32),
            mesh=_mesh(),
            scratch_types=[
                pltpu.VMEM((NCHUNK, CH), jnp.int32),
                pltpu.VMEM((NCHUNK, CH), jnp.int32),
                pltpu.VMEM((NBUF, CH, DH), jnp.float32),
                pltpu.VMEM((CH, DH), jnp.float32),
                pltpu.VMEM_SHARED((NP, DH), jnp.float32),
                pltpu.VMEM_SHARED((NP, DH), jnp.float32),
            ] + [pltpu.SemaphoreType.DMA] * (2 * NBUF),
            compiler_params=pltpu.CompilerParams(use_tc_tiling_on_sc=False),
        )
    return _SC_KERNELS["scat"](y, src_r, dst_r)


def _dis_block(degb, i):
    deg = degb[0, :, 0:1] + degb[1, :, 0:1] + 1.0
    return lax.rsqrt(deg)


def _row_mask(i, val):
    row = i * BR + lax.broadcasted_iota(jnp.int32, val.shape, 0)
    return jnp.where(row < N, val, 0.0)


def _tc_first_body(x_ref, w_ref, deg_ref, y_ref):
    i = pl.program_id(0)
    dis = _dis_block(deg_ref[...], i)
    xw = jnp.dot(x_ref[...], w_ref[...], preferred_element_type=jnp.float32)
    y_ref[...] = _row_mask(i, dis * xw)


def _tc_mid_body(acc_ref, y_ref, deg_ref, b_ref, w_ref, out_ref):
    i = pl.program_id(0)
    dis = _dis_block(deg_ref[...], i)
    t = acc_ref[0] + acc_ref[1] + y_ref[...]
    h = dis * t + b_ref[...]
    h = jnp.where(h >= 0, h, NEG_SLOPE * h)
    y = dis * jnp.dot(h, w_ref[...], preferred_element_type=jnp.float32)
    out_ref[...] = _row_mask(i, y)


def _tc_fin_body(acc_ref, y_ref, deg_ref, b_ref, out_ref):
    i = pl.program_id(0)
    dis = _dis_block(deg_ref[...], i)
    t = acc_ref[0] + acc_ref[1] + y_ref[...]
    h = dis * t + b_ref[...]
    out_ref[...] = jnp.where(h >= 0, h, NEG_SLOPE * h)


_GRID = NP // BR

_spec_deg = pl.BlockSpec((2, BR, 16), lambda i: (0, i, 0))
_spec_acc = pl.BlockSpec((2, BR, DH), lambda i: (0, i, 0))
_spec_row64 = pl.BlockSpec((BR, DH), lambda i: (i, 0))
_spec_b = pl.BlockSpec((1, DH), lambda i: (0, 0))


def _tc_first(x_pad, W1, deg_p):
    return pl.pallas_call(
        _tc_first_body,
        grid=(_GRID,),
        in_specs=[pl.BlockSpec((BR, D_IN), lambda i: (i, 0)),
                  pl.BlockSpec((D_IN, DH), lambda i: (0, 0)),
                  _spec_deg],
        out_specs=_spec_row64,
        out_shape=jax.ShapeDtypeStruct((NP, DH), jnp.float32),
    )(x_pad, W1, deg_p)


def _tc_mid(acc_p, y_prev, deg_p, b_prev, W_next):
    return pl.pallas_call(
        _tc_mid_body,
        grid=(_GRID,),
        in_specs=[_spec_acc, _spec_row64, _spec_deg, _spec_b,
                  pl.BlockSpec((DH, DH), lambda i: (0, 0))],
        out_specs=_spec_row64,
        out_shape=jax.ShapeDtypeStruct((NP, DH), jnp.float32),
    )(acc_p, y_prev, deg_p, b_prev, W_next)


def _tc_fin(acc_p, y_prev, deg_p, b_prev):
    return pl.pallas_call(
        _tc_fin_body,
        grid=(_GRID,),
        in_specs=[_spec_acc, _spec_row64, _spec_deg, _spec_b],
        out_specs=_spec_row64,
        out_shape=jax.ShapeDtypeStruct((NP, DH), jnp.float32),
    )(acc_p, y_prev, deg_p, b_prev)


def kernel(x, edge_index, batch, W1, b1, W2, b2, W3, b3):
    src = edge_index[0]
    dst = edge_index[1]
    pad = jnp.full((E_PAD - E,), N, jnp.int32)
    src_r = jnp.concatenate([src, pad]).reshape(NW, NCHUNK, CH)
    dst_r = jnp.concatenate([dst, pad]).reshape(NW, NCHUNK, CH)
    x_pad = jnp.pad(x, ((0, NP - N), (0, 0)))
    b1r = b1.reshape(1, DH)
    b2r = b2.reshape(1, DH)
    b3r = b3.reshape(1, DH)

    deg_p = _sc_degree(dst_r)
    y1 = _tc_first(x_pad, W1, deg_p)
    acc1 = _sc_scatter(y1, src_r, dst_r)
    y2 = _tc_mid(acc1, y1, deg_p, b1r, W2)
    acc2 = _sc_scatter(y2, src_r, dst_r)
    y3 = _tc_mid(acc2, y2, deg_p, b2r, W3)
    acc3 = _sc_scatter(y3, src_r, dst_r)
    h3 = _tc_fin(acc3, y3, deg_p, b3r)
    return h3[:N]


# speedup vs baseline: 1.2158x; 1.2158x over previous
import jax
import jax.numpy as jnp
from jax import lax
from jax.experimental import pallas as pl
from jax.experimental.pallas import tpu as pltpu
from jax.experimental.pallas import tpu_sc as plsc

N = 10000
E = 320000
D_IN = 128
DH = 64
NEG_SLOPE = 0.01

NC = 2
NS = 16
NW = NC * NS
CH = 128
EB = E // CH
NCHB = EB // NW
NXTRA = EB - NCHB * NW
NCHMAX = NCHB + 1
NP = 10112
ROWS_PER_TILE = NP // NS
GRID = 4
BR = NP // GRID

_MESH_CACHE = []


def _mesh():
    if not _MESH_CACHE:
        _MESH_CACHE.append(
            plsc.VectorSubcoreMesh(core_axis_name="c", subcore_axis_name="s",
                                   num_cores=NC, num_subcores=NS))
    return _MESH_CACHE[0]


def _zero_fill(buf, nrows, width):
    zv = jnp.zeros((16,), jnp.float32)

    def body(t, _):
        i = t // (width // 16)
        k = t % (width // 16)
        buf[i, pl.ds(k * 16, 16)] = zv
        return 0

    lax.fori_loop(0, nrows * (width // 16), body, 0)


def _ones_fill(buf, nrows, width):
    ov = jnp.ones((16,), jnp.float32)

    def body(t, _):
        i = t // (width // 16)
        k = t % (width // 16)
        buf[i, pl.ds(k * 16, 16)] = ov
        return 0

    lax.fori_loop(0, nrows * (width // 16), body, 0)


def _zero_stripe(acc_sp, z_v, base):
    nfull = ROWS_PER_TILE // CH
    rem = ROWS_PER_TILE - nfull * CH

    def body(k, _):
        pltpu.sync_copy(z_v, acc_sp.at[pl.ds(base + k * CH, CH)])
        return 0

    lax.fori_loop(0, nfull, body, 0)
    if rem:
        pltpu.sync_copy(z_v.at[pl.ds(0, rem)],
                        acc_sp.at[pl.ds(base + nfull * CH, rem)])


def _work_range(wid):
    n = NCHB + jnp.where(wid < NXTRA, 1, 0)
    start = NCHB * wid + jnp.minimum(wid, NXTRA)
    return start, n


def _load_idx(ev_hbm, sel, start, wid, idx_v):
    pltpu.sync_copy(ev_hbm.at[pl.ds(start, NCHB), sel],
                    idx_v.at[pl.ds(0, NCHB)])

    @pl.when(wid < NXTRA)
    def _():
        pltpu.sync_copy(ev_hbm.at[start + NCHB, sel], idx_v.at[NCHB])


def _sc_degree_body(ev_hbm, out_hbm, idx_v, ones_v, z_v, deg_sp):
    cid = lax.axis_index("c")
    sid = lax.axis_index("s")
    wid = sid * NC + cid
    base = sid * ROWS_PER_TILE
    start, n = _work_range(wid)

    _ones_fill(ones_v, CH, 16)
    _zero_fill(z_v, CH, 16)
    _zero_stripe(deg_sp, z_v, base)
    _load_idx(ev_hbm, 1, start, wid, idx_v)
    plsc.subcore_barrier()

    def body(j, _):
        pltpu.sync_copy(ones_v, deg_sp.at[idx_v.at[j]], add=True)
        return 0

    lax.fori_loop(0, n, body, 0)
    plsc.subcore_barrier()
    pltpu.sync_copy(deg_sp.at[pl.ds(base, ROWS_PER_TILE)],
                    out_hbm.at[cid, pl.ds(base, ROWS_PER_TILE), pl.ds(0, 16)])


def _sc_scatter_body(y_hbm, ev_hbm, out_hbm,
                     src_v, dst_v, rows_v, z_v, acc_sp, y_sp, sem0, sem1):
    cid = lax.axis_index("c")
    sid = lax.axis_index("s")
    wid = sid * NC + cid
    base = sid * ROWS_PER_TILE
    start, n = _work_range(wid)

    pltpu.async_copy(y_hbm.at[pl.ds(base, ROWS_PER_TILE), pl.ds(0, DH)],
                     y_sp.at[pl.ds(base, ROWS_PER_TILE)], sem1)
    _zero_fill(z_v, CH, DH)
    _zero_stripe(acc_sp, z_v, base)
    _load_idx(ev_hbm, 0, start, wid, src_v)
    _load_idx(ev_hbm, 1, start, wid, dst_v)
    pltpu.make_async_copy(y_hbm.at[pl.ds(base, ROWS_PER_TILE), pl.ds(0, DH)],
                          y_sp.at[pl.ds(base, ROWS_PER_TILE)], sem1).wait()
    plsc.subcore_barrier()

    pltpu.async_copy(y_sp.at[src_v.at[0]], rows_v.at[0], sem0)

    def body(j, _):
        cur = j % 2

        @pl.when(cur == 0)
        def _():
            pltpu.make_async_copy(y_sp.at[src_v.at[j]], rows_v.at[0],
                                  sem0).wait()

        @pl.when(cur == 1)
        def _():
            pltpu.make_async_copy(y_sp.at[src_v.at[j]], rows_v.at[1],
                                  sem1).wait()

        @pl.when(j + 1 < n)
        def _():
            nxt = (j + 1) % 2

            @pl.when(nxt == 0)
            def _():
                pltpu.async_copy(y_sp.at[src_v.at[j + 1]], rows_v.at[0], sem0)

            @pl.when(nxt == 1)
            def _():
                pltpu.async_copy(y_sp.at[src_v.at[j + 1]], rows_v.at[1], sem1)

        pltpu.sync_copy(rows_v.at[cur], acc_sp.at[dst_v.at[j]], add=True)
        return 0

    lax.fori_loop(0, n, body, 0)
    plsc.subcore_barrier()
    pltpu.sync_copy(acc_sp.at[pl.ds(base, ROWS_PER_TILE)],
                    out_hbm.at[cid, pl.ds(base, ROWS_PER_TILE), pl.ds(0, DH)])


_SC_KERNELS = {}


def _sc_degree(ev):
    if "deg" not in _SC_KERNELS:
        _SC_KERNELS["deg"] = pl.kernel(
            _sc_degree_body,
            out_type=jax.ShapeDtypeStruct((NC, NP, 128), jnp.float32),
            mesh=_mesh(),
            scratch_types=[
                pltpu.VMEM((NCHMAX, CH), jnp.int32),
                pltpu.VMEM((CH, 16), jnp.float32),
                pltpu.VMEM((CH, 16), jnp.float32),
                pltpu.VMEM_SHARED((NP, 16), jnp.float32),
            ],
            compiler_params=pltpu.CompilerParams(use_tc_tiling_on_sc=False),
        )
    return _SC_KERNELS["deg"](ev)


def _sc_scatter(y, ev):
    if "scat" not in _SC_KERNELS:
        _SC_KERNELS["scat"] = pl.kernel(
            _sc_scatter_body,
            out_type=jax.ShapeDtypeStruct((NC, NP, 128), jnp.float32),
            mesh=_mesh(),
            scratch_types=[
                pltpu.VMEM((NCHMAX, CH), jnp.int32),
                pltpu.VMEM((NCHMAX, CH), jnp.int32),
                pltpu.VMEM((2, CH, DH), jnp.float32),
                pltpu.VMEM((CH, DH), jnp.float32),
                pltpu.VMEM_SHARED((NP, DH), jnp.float32),
                pltpu.VMEM_SHARED((NP, DH), jnp.float32),
                pltpu.SemaphoreType.DMA,
                pltpu.SemaphoreType.DMA,
            ],
            compiler_params=pltpu.CompilerParams(use_tc_tiling_on_sc=False),
        )
    return _SC_KERNELS["scat"](y, ev)


def _dis_block(degb):
    deg = degb[0, :, 0:1] + degb[1, :, 0:1] + 1.0
    return lax.rsqrt(deg)


def _pad128(v):
    return jnp.concatenate([v, jnp.zeros_like(v)], axis=1)


def _tc_first_body(x_ref, w_ref, deg_ref, y_ref):
    dis = _dis_block(deg_ref[...])
    xw = jnp.dot(x_ref[...], w_ref[...], preferred_element_type=jnp.float32)
    y_ref[...] = _pad128(dis * xw)


def _tc_mid_body(acc_ref, y_ref, deg_ref, b_ref, w_ref, out_ref):
    dis = _dis_block(deg_ref[...])
    t = acc_ref[0, :, :DH] + acc_ref[1, :, :DH] + y_ref[:, :DH]
    h = dis * t + b_ref[...]
    h = jnp.where(h >= 0, h, NEG_SLOPE * h)
    out_ref[...] = _pad128(dis * jnp.dot(h, w_ref[...],
                                         preferred_element_type=jnp.float32))


def _tc_fin_body(acc_ref, y_ref, deg_ref, b_ref, out_ref):
    dis = _dis_block(deg_ref[...])
    t = acc_ref[0, :, :DH] + acc_ref[1, :, :DH] + y_ref[:, :DH]
    h = dis * t + b_ref[...]
    out_ref[...] = jnp.where(h >= 0, h, NEG_SLOPE * h)


_spec_deg = pl.BlockSpec((2, BR, 128), lambda i: (0, i, 0))
_spec_acc = pl.BlockSpec((2, BR, 128), lambda i: (0, i, 0))
_spec_row64 = pl.BlockSpec((BR, 128), lambda i: (i, 0))
_spec_b = pl.BlockSpec((1, DH), lambda i: (0, 0))


def _tc_first(x, W1, deg_p):
    return pl.pallas_call(
        _tc_first_body,
        grid=(GRID,),
        in_specs=[pl.BlockSpec((BR, D_IN), lambda i: (i, 0)),
                  pl.BlockSpec((D_IN, DH), lambda i: (0, 0)),
                  _spec_deg],
        out_specs=_spec_row64,
        out_shape=jax.ShapeDtypeStruct((NP, 128), jnp.float32),
    )(x, W1, deg_p)


def _tc_mid(acc_p, y_prev, deg_p, b_prev, W_next):
    return pl.pallas_call(
        _tc_mid_body,
        grid=(GRID,),
        in_specs=[_spec_acc, _spec_row64, _spec_deg, _spec_b,
                  pl.BlockSpec((DH, DH), lambda i: (0, 0))],
        out_specs=_spec_row64,
        out_shape=jax.ShapeDtypeStruct((NP, 128), jnp.float32),
    )(acc_p, y_prev, deg_p, b_prev, W_next)


def _tc_fin(acc_p, y_prev, deg_p, b_prev):
    return pl.pallas_call(
        _tc_fin_body,
        grid=(GRID,),
        in_specs=[_spec_acc, _spec_row64, _spec_deg, _spec_b],
        out_specs=pl.BlockSpec((BR, DH), lambda i: (i, 0)),
        out_shape=jax.ShapeDtypeStruct((N, DH), jnp.float32),
    )(acc_p, y_prev, deg_p, b_prev)


def kernel(x, edge_index, batch, W1, b1, W2, b2, W3, b3):
    ev = jnp.transpose(edge_index.reshape(2, EB, CH), (1, 0, 2))
    b1r = b1.reshape(1, DH)
    b2r = b2.reshape(1, DH)
    b3r = b3.reshape(1, DH)

    deg_p = _sc_degree(ev)
    y1 = _tc_first(x, W1, deg_p)
    acc1 = _sc_scatter(y1, ev)
    y2 = _tc_mid(acc1, y1, deg_p, b1r, W2)
    acc2 = _sc_scatter(y2, ev)
    y3 = _tc_mid(acc2, y2, deg_p, b2r, W3)
    acc3 = _sc_scatter(y3, ev)
    return _tc_fin(acc3, y3, deg_p, b3r)


# speedup vs baseline: 1.2737x; 1.0476x over previous
import jax
import jax.numpy as jnp
from jax import lax
from jax.experimental import pallas as pl
from jax.experimental.pallas import tpu as pltpu
from jax.experimental.pallas import tpu_sc as plsc

N = 10000
E = 320000
D_IN = 128
DH = 64
NEG_SLOPE = 0.01

NC = 2
NS = 16
NW = NC * NS
CH = 128
EB = E // CH
NCHB = EB // NW
NXTRA = EB - NCHB * NW
NCHMAX = NCHB + 1
NP = 10112
ROWS_PER_TILE = NP // NS
GRID = 4
BR = 2560

_MESH_CACHE = []


def _mesh():
    if not _MESH_CACHE:
        _MESH_CACHE.append(
            plsc.VectorSubcoreMesh(core_axis_name="c", subcore_axis_name="s",
                                   num_cores=NC, num_subcores=NS))
    return _MESH_CACHE[0]


def _zero_fill(buf, nrows, width):
    zv = jnp.zeros((16,), jnp.float32)

    def body(t, _):
        i = t // (width // 16)
        k = t % (width // 16)
        buf[i, pl.ds(k * 16, 16)] = zv
        return 0

    lax.fori_loop(0, nrows * (width // 16), body, 0)


def _ones_fill(buf, nrows, width):
    ov = jnp.ones((16,), jnp.float32)

    def body(t, _):
        i = t // (width // 16)
        k = t % (width // 16)
        buf[i, pl.ds(k * 16, 16)] = ov
        return 0

    lax.fori_loop(0, nrows * (width // 16), body, 0)


def _zero_stripe(acc_sp, z_v, base):
    nfull = ROWS_PER_TILE // CH
    rem = ROWS_PER_TILE - nfull * CH

    def body(k, _):
        pltpu.sync_copy(z_v, acc_sp.at[pl.ds(base + k * CH, CH)])
        return 0

    lax.fori_loop(0, nfull, body, 0)
    if rem:
        pltpu.sync_copy(z_v.at[pl.ds(0, rem)],
                        acc_sp.at[pl.ds(base + nfull * CH, rem)])


def _work_range(wid):
    n = NCHB + jnp.where(wid < NXTRA, 1, 0)
    start = NCHB * wid + jnp.minimum(wid, NXTRA)
    return start, n


def _load_idx(ev_hbm, sel, start, wid, idx_v):
    pltpu.sync_copy(ev_hbm.at[pl.ds(start, NCHB), sel],
                    idx_v.at[pl.ds(0, NCHB)])

    @pl.when(wid < NXTRA)
    def _():
        pltpu.sync_copy(ev_hbm.at[start + NCHB, sel], idx_v.at[NCHB])


def _sc_degree_body(ev_hbm, out_hbm, idx_v, ones_v, z_v, deg_sp):
    cid = lax.axis_index("c")
    sid = lax.axis_index("s")
    wid = sid * NC + cid
    base = sid * ROWS_PER_TILE
    start, n = _work_range(wid)

    _ones_fill(ones_v, CH, 16)
    _zero_fill(z_v, CH, 16)
    _zero_stripe(deg_sp, z_v, base)
    _load_idx(ev_hbm, 1, start, wid, idx_v)
    plsc.subcore_barrier()

    def body(j, _):
        pltpu.sync_copy(ones_v, deg_sp.at[idx_v.at[j]], add=True)
        return 0

    lax.fori_loop(0, n, body, 0)
    plsc.subcore_barrier()
    pltpu.sync_copy(deg_sp.at[pl.ds(base, ROWS_PER_TILE)],
                    out_hbm.at[cid, pl.ds(base, ROWS_PER_TILE), pl.ds(0, 16)])


def _sc_scatter_body(y_hbm, ev_hbm, out_hbm,
                     src_v, dst_v, rows_v, z_v, acc_sp, y_sp, sem0, sem1):
    cid = lax.axis_index("c")
    sid = lax.axis_index("s")
    wid = sid * NC + cid
    base = sid * ROWS_PER_TILE
    start, n = _work_range(wid)

    pltpu.async_copy(y_hbm.at[pl.ds(base, ROWS_PER_TILE), pl.ds(0, DH)],
                     y_sp.at[pl.ds(base, ROWS_PER_TILE)], sem1)
    _zero_fill(z_v, CH, DH)
    _zero_stripe(acc_sp, z_v, base)
    _load_idx(ev_hbm, 0, start, wid, src_v)
    _load_idx(ev_hbm, 1, start, wid, dst_v)
    pltpu.make_async_copy(y_hbm.at[pl.ds(base, ROWS_PER_TILE), pl.ds(0, DH)],
                          y_sp.at[pl.ds(base, ROWS_PER_TILE)], sem1).wait()
    plsc.subcore_barrier()

    pltpu.async_copy(y_sp.at[src_v.at[0]], rows_v.at[0], sem0)

    def body(j, _):
        cur = j % 2

        @pl.when(cur == 0)
        def _():
            pltpu.make_async_copy(y_sp.at[src_v.at[j]], rows_v.at[0],
                                  sem0).wait()

        @pl.when(cur == 1)
        def _():
            pltpu.make_async_copy(y_sp.at[src_v.at[j]], rows_v.at[1],
                                  sem1).wait()

        @pl.when(j + 1 < n)
        def _():
            nxt = (j + 1) % 2

            @pl.when(nxt == 0)
            def _():
                pltpu.async_copy(y_sp.at[src_v.at[j + 1]], rows_v.at[0], sem0)

            @pl.when(nxt == 1)
            def _():
                pltpu.async_copy(y_sp.at[src_v.at[j + 1]], rows_v.at[1], sem1)

        pltpu.sync_copy(rows_v.at[cur], acc_sp.at[dst_v.at[j]], add=True)
        return 0

    lax.fori_loop(0, n, body, 0)
    plsc.subcore_barrier()
    pltpu.sync_copy(acc_sp.at[pl.ds(base, ROWS_PER_TILE)],
                    out_hbm.at[cid, pl.ds(base, ROWS_PER_TILE), pl.ds(0, DH)])


_SC_KERNELS = {}


def _sc_degree(ev):
    if "deg" not in _SC_KERNELS:
        _SC_KERNELS["deg"] = pl.kernel(
            _sc_degree_body,
            out_type=jax.ShapeDtypeStruct((NC, NP, 128), jnp.float32),
            mesh=_mesh(),
            scratch_types=[
                pltpu.VMEM((NCHMAX, CH), jnp.int32),
                pltpu.VMEM((CH, 16), jnp.float32),
                pltpu.VMEM((CH, 16), jnp.float32),
                pltpu.VMEM_SHARED((NP, 16), jnp.float32),
            ],
            compiler_params=pltpu.CompilerParams(use_tc_tiling_on_sc=False),
        )
    return _SC_KERNELS["deg"](ev)


def _sc_scatter(y, ev):
    if "scat" not in _SC_KERNELS:
        _SC_KERNELS["scat"] = pl.kernel(
            _sc_scatter_body,
            out_type=jax.ShapeDtypeStruct((NC, NP, 128), jnp.float32),
            mesh=_mesh(),
            scratch_types=[
                pltpu.VMEM((NCHMAX, CH), jnp.int32),
                pltpu.VMEM((NCHMAX, CH), jnp.int32),
                pltpu.VMEM((2, CH, DH), jnp.float32),
                pltpu.VMEM((CH, DH), jnp.float32),
                pltpu.VMEM_SHARED((NP, DH), jnp.float32),
                pltpu.VMEM_SHARED((NP, DH), jnp.float32),
                pltpu.SemaphoreType.DMA,
                pltpu.SemaphoreType.DMA,
            ],
            compiler_params=pltpu.CompilerParams(use_tc_tiling_on_sc=False),
        )
    return _SC_KERNELS["scat"](y, ev)


def _dis_block(degb):
    deg = degb[0, :, 0:1] + degb[1, :, 0:1] + 1.0
    return lax.rsqrt(deg)


def _with_dis(v, dis):
    return jnp.concatenate([v, jnp.broadcast_to(dis, v.shape)], axis=1)


def _tc_first_body(x_ref, w_ref, deg_ref, y_ref):
    dis = _dis_block(deg_ref[...])
    xw = jnp.dot(x_ref[...], w_ref[...], preferred_element_type=jnp.float32)
    y_ref[...] = _with_dis(dis * xw, dis)


def _tc_mid_body(acc_ref, y_ref, b_ref, w_ref, out_ref):
    dis = y_ref[:, DH:DH + 1]
    t = acc_ref[0, :, :DH] + acc_ref[1, :, :DH] + y_ref[:, :DH]
    h = dis * t + b_ref[...]
    h = jnp.where(h >= 0, h, NEG_SLOPE * h)
    out_ref[...] = _with_dis(dis * jnp.dot(h, w_ref[...],
                                           preferred_element_type=jnp.float32),
                             dis)


def _tc_fin_body(acc_ref, y_ref, b_ref, out_ref):
    dis = y_ref[:, DH:DH + 1]
    t = acc_ref[0, :, :DH] + acc_ref[1, :, :DH] + y_ref[:, :DH]
    h = dis * t + b_ref[...]
    h = jnp.where(h >= 0, h, NEG_SLOPE * h)
    out_ref[...] = h.T


_spec_deg = pl.BlockSpec((2, BR, 128), lambda i: (0, i, 0))
_spec_acc = pl.BlockSpec((2, BR, 128), lambda i: (0, i, 0))
_spec_row64 = pl.BlockSpec((BR, 128), lambda i: (i, 0))
_spec_b = pl.BlockSpec((1, DH), lambda i: (0, 0))


def _tc_first(x, W1, deg_p):
    return pl.pallas_call(
        _tc_first_body,
        grid=(GRID,),
        in_specs=[pl.BlockSpec((BR, D_IN), lambda i: (i, 0)),
                  pl.BlockSpec((D_IN, DH), lambda i: (0, 0)),
                  _spec_deg],
        out_specs=_spec_row64,
        out_shape=jax.ShapeDtypeStruct((NP, 128), jnp.float32),
    )(x, W1, deg_p)


def _tc_mid(acc_p, y_prev, b_prev, W_next):
    return pl.pallas_call(
        _tc_mid_body,
        grid=(GRID,),
        in_specs=[_spec_acc, _spec_row64, _spec_b,
                  pl.BlockSpec((DH, DH), lambda i: (0, 0))],
        out_specs=_spec_row64,
        out_shape=jax.ShapeDtypeStruct((NP, 128), jnp.float32),
    )(acc_p, y_prev, b_prev, W_next)


def _tc_fin(acc_p, y_prev, b_prev):
    return pl.pallas_call(
        _tc_fin_body,
        grid=(GRID,),
        in_specs=[_spec_acc, _spec_row64, _spec_b],
        out_specs=pl.BlockSpec((DH, BR), lambda i: (0, i)),
        out_shape=jax.ShapeDtypeStruct((DH, N), jnp.float32),
    )(acc_p, y_prev, b_prev)


def kernel(x, edge_index, batch, W1, b1, W2, b2, W3, b3):
    ev = jnp.transpose(edge_index.reshape(2, EB, CH), (1, 0, 2))
    b1r = b1.reshape(1, DH)
    b2r = b2.reshape(1, DH)
    b3r = b3.reshape(1, DH)

    deg_p = _sc_degree(ev)
    y1 = _tc_first(x, W1, deg_p)
    acc1 = _sc_scatter(y1, ev)
    y2 = _tc_mid(acc1, y1, b1r, W2)
    acc2 = _sc_scatter(y2, ev)
    y3 = _tc_mid(acc2, y2, b2r, W3)
    acc3 = _sc_scatter(y3, ev)
    return _tc_fin(acc3, y3, b3r).T


# speedup vs baseline: 1.2861x; 1.0098x over previous
import jax
import jax.numpy as jnp
from jax import lax
from jax.experimental import pallas as pl
from jax.experimental.pallas import tpu as pltpu
from jax.experimental.pallas import tpu_sc as plsc

N = 10000
E = 320000
D_IN = 128
DH = 64
NEG_SLOPE = 0.01

NC = 2
NS = 16
NW = NC * NS
CH = 128
EB = E // CH
NCHB = EB // NW
NXTRA = EB - NCHB * NW
NCHMAX = NCHB + 1
NP = 10112
ROWS_PER_TILE = NP // NS
GRID = 2
BR = 5120

_MESH_CACHE = []


def _mesh():
    if not _MESH_CACHE:
        _MESH_CACHE.append(
            plsc.VectorSubcoreMesh(core_axis_name="c", subcore_axis_name="s",
                                   num_cores=NC, num_subcores=NS))
    return _MESH_CACHE[0]


def _zero_fill(buf, nrows, width):
    zv = jnp.zeros((16,), jnp.float32)

    def body(t, _):
        i = t // (width // 16)
        k = t % (width // 16)
        buf[i, pl.ds(k * 16, 16)] = zv
        return 0

    lax.fori_loop(0, nrows * (width // 16), body, 0)


def _ones_fill(buf, nrows, width):
    ov = jnp.ones((16,), jnp.float32)

    def body(t, _):
        i = t // (width // 16)
        k = t % (width // 16)
        buf[i, pl.ds(k * 16, 16)] = ov
        return 0

    lax.fori_loop(0, nrows * (width // 16), body, 0)


def _zero_stripe(acc_sp, z_v, base):
    nfull = ROWS_PER_TILE // CH
    rem = ROWS_PER_TILE - nfull * CH

    def body(k, _):
        pltpu.sync_copy(z_v, acc_sp.at[pl.ds(base + k * CH, CH)])
        return 0

    lax.fori_loop(0, nfull, body, 0)
    if rem:
        pltpu.sync_copy(z_v.at[pl.ds(0, rem)],
                        acc_sp.at[pl.ds(base + nfull * CH, rem)])


def _work_range(wid):
    n = NCHB + jnp.where(wid < NXTRA, 1, 0)
    start = NCHB * wid + jnp.minimum(wid, NXTRA)
    return start, n


def _load_idx(ev_hbm, sel, start, wid, idx_v):
    pltpu.sync_copy(ev_hbm.at[pl.ds(start, NCHB), sel],
                    idx_v.at[pl.ds(0, NCHB)])

    @pl.when(wid < NXTRA)
    def _():
        pltpu.sync_copy(ev_hbm.at[start + NCHB, sel], idx_v.at[NCHB])


def _sc_degree_body(ev_hbm, out_hbm, idx_v, ones_v, z_v, deg_sp):
    cid = lax.axis_index("c")
    sid = lax.axis_index("s")
    wid = sid * NC + cid
    base = sid * ROWS_PER_TILE
    start, n = _work_range(wid)

    _ones_fill(ones_v, CH, 16)
    _zero_fill(z_v, CH, 16)
    _zero_stripe(deg_sp, z_v, base)
    _load_idx(ev_hbm, 1, start, wid, idx_v)
    plsc.subcore_barrier()

    def body(j, _):
        pltpu.sync_copy(ones_v, deg_sp.at[idx_v.at[j]], add=True)
        return 0

    lax.fori_loop(0, n, body, 0)
    plsc.subcore_barrier()
    pltpu.sync_copy(deg_sp.at[pl.ds(base, ROWS_PER_TILE)],
                    out_hbm.at[cid, pl.ds(base, ROWS_PER_TILE), pl.ds(0, 16)])


def _sc_scatter_body(y_hbm, ev_hbm, out_hbm,
                     src_v, dst_v, rows_v, z_v, acc_sp, y_sp, sem0, sem1):
    cid = lax.axis_index("c")
    sid = lax.axis_index("s")
    wid = sid * NC + cid
    base = sid * ROWS_PER_TILE
    start, n = _work_range(wid)

    pltpu.async_copy(y_hbm.at[pl.ds(base, ROWS_PER_TILE), pl.ds(0, DH)],
                     y_sp.at[pl.ds(base, ROWS_PER_TILE)], sem1)
    _zero_fill(z_v, CH, DH)
    _zero_stripe(acc_sp, z_v, base)
    _load_idx(ev_hbm, 0, start, wid, src_v)
    _load_idx(ev_hbm, 1, start, wid, dst_v)
    pltpu.make_async_copy(y_hbm.at[pl.ds(base, ROWS_PER_TILE), pl.ds(0, DH)],
                          y_sp.at[pl.ds(base, ROWS_PER_TILE)], sem1).wait()
    plsc.subcore_barrier()

    pltpu.async_copy(y_sp.at[src_v.at[0]], rows_v.at[0], sem0)

    def body(j, _):
        cur = j % 2

        @pl.when(cur == 0)
        def _():
            pltpu.make_async_copy(y_sp.at[src_v.at[j]], rows_v.at[0],
                                  sem0).wait()

        @pl.when(cur == 1)
        def _():
            pltpu.make_async_copy(y_sp.at[src_v.at[j]], rows_v.at[1],
                                  sem1).wait()

        @pl.when(j + 1 < n)
        def _():
            nxt = (j + 1) % 2

            @pl.when(nxt == 0)
            def _():
                pltpu.async_copy(y_sp.at[src_v.at[j + 1]], rows_v.at[0], sem0)

            @pl.when(nxt == 1)
            def _():
                pltpu.async_copy(y_sp.at[src_v.at[j + 1]], rows_v.at[1], sem1)

        pltpu.sync_copy(rows_v.at[cur], acc_sp.at[dst_v.at[j]], add=True)
        return 0

    lax.fori_loop(0, n, body, 0)
    plsc.subcore_barrier()
    pltpu.sync_copy(acc_sp.at[pl.ds(base, ROWS_PER_TILE)],
                    out_hbm.at[cid, pl.ds(base, ROWS_PER_TILE), pl.ds(0, DH)])


_SC_KERNELS = {}


def _sc_degree(ev):
    if "deg" not in _SC_KERNELS:
        _SC_KERNELS["deg"] = pl.kernel(
            _sc_degree_body,
            out_type=jax.ShapeDtypeStruct((NC, NP, 128), jnp.float32),
            mesh=_mesh(),
            scratch_types=[
                pltpu.VMEM((NCHMAX, CH), jnp.int32),
                pltpu.VMEM((CH, 16), jnp.float32),
                pltpu.VMEM((CH, 16), jnp.float32),
                pltpu.VMEM_SHARED((NP, 16), jnp.float32),
            ],
            compiler_params=pltpu.CompilerParams(use_tc_tiling_on_sc=False),
        )
    return _SC_KERNELS["deg"](ev)


def _sc_scatter(y, ev):
    if "scat" not in _SC_KERNELS:
        _SC_KERNELS["scat"] = pl.kernel(
            _sc_scatter_body,
            out_type=jax.ShapeDtypeStruct((NC, NP, 128), jnp.float32),
            mesh=_mesh(),
            scratch_types=[
                pltpu.VMEM((NCHMAX, CH), jnp.int32),
                pltpu.VMEM((NCHMAX, CH), jnp.int32),
                pltpu.VMEM((2, CH, DH), jnp.float32),
                pltpu.VMEM((CH, DH), jnp.float32),
                pltpu.VMEM_SHARED((NP, DH), jnp.float32),
                pltpu.VMEM_SHARED((NP, DH), jnp.float32),
                pltpu.SemaphoreType.DMA,
                pltpu.SemaphoreType.DMA,
            ],
            compiler_params=pltpu.CompilerParams(use_tc_tiling_on_sc=False),
        )
    return _SC_KERNELS["scat"](y, ev)


def _dis_block(degb):
    deg = degb[0, :, 0:1] + degb[1, :, 0:1] + 1.0
    return lax.rsqrt(deg)


def _with_dis(v, dis):
    return jnp.concatenate([v, jnp.broadcast_to(dis, v.shape)], axis=1)


def _tc_first_body(x_ref, w_ref, deg_ref, y_ref):
    dis = _dis_block(deg_ref[...])
    xw = jnp.dot(x_ref[...], w_ref[...], preferred_element_type=jnp.float32)
    y_ref[...] = _with_dis(dis * xw, dis)


def _tc_mid_body(acc_ref, y_ref, b_ref, w_ref, out_ref):
    dis = y_ref[:, DH:DH + 1]
    t = acc_ref[0, :, :DH] + acc_ref[1, :, :DH] + y_ref[:, :DH]
    h = dis * t + b_ref[...]
    h = jnp.where(h >= 0, h, NEG_SLOPE * h)
    out_ref[...] = _with_dis(dis * jnp.dot(h, w_ref[...],
                                           preferred_element_type=jnp.float32),
                             dis)


def _tc_fin_body(acc_ref, y_ref, b_ref, out_ref):
    dis = y_ref[:, DH:DH + 1]
    t = acc_ref[0, :, :DH] + acc_ref[1, :, :DH] + y_ref[:, :DH]
    h = dis * t + b_ref[...]
    h = jnp.where(h >= 0, h, NEG_SLOPE * h)
    out_ref[...] = h.T


_spec_deg = pl.BlockSpec((2, BR, 128), lambda i: (0, i, 0))
_spec_acc = pl.BlockSpec((2, BR, 128), lambda i: (0, i, 0))
_spec_row64 = pl.BlockSpec((BR, 128), lambda i: (i, 0))
_spec_b = pl.BlockSpec((1, DH), lambda i: (0, 0))


def _tc_first(x, W1, deg_p):
    return pl.pallas_call(
        _tc_first_body,
        grid=(GRID,),
        in_specs=[pl.BlockSpec((BR, D_IN), lambda i: (i, 0)),
                  pl.BlockSpec((D_IN, DH), lambda i: (0, 0)),
                  _spec_deg],
        out_specs=_spec_row64,
        out_shape=jax.ShapeDtypeStruct((NP, 128), jnp.float32),
    )(x, W1, deg_p)


def _tc_mid(acc_p, y_prev, b_prev, W_next):
    return pl.pallas_call(
        _tc_mid_body,
        grid=(GRID,),
        in_specs=[_spec_acc, _spec_row64, _spec_b,
                  pl.BlockSpec((DH, DH), lambda i: (0, 0))],
        out_specs=_spec_row64,
        out_shape=jax.ShapeDtypeStruct((NP, 128), jnp.float32),
    )(acc_p, y_prev, b_prev, W_next)


def _tc_fin(acc_p, y_prev, b_prev):
    return pl.pallas_call(
        _tc_fin_body,
        grid=(GRID,),
        in_specs=[_spec_acc, _spec_row64, _spec_b],
        out_specs=pl.BlockSpec((DH, BR), lambda i: (0, i)),
        out_shape=jax.ShapeDtypeStruct((DH, N), jnp.float32),
    )(acc_p, y_prev, b_prev)


def kernel(x, edge_index, batch, W1, b1, W2, b2, W3, b3):
    ev = jnp.transpose(edge_index.reshape(2, EB, CH), (1, 0, 2))
    b1r = b1.reshape(1, DH)
    b2r = b2.reshape(1, DH)
    b3r = b3.reshape(1, DH)

    deg_p = _sc_degree(ev)
    y1 = _tc_first(x, W1, deg_p)
    acc1 = _sc_scatter(y1, ev)
    y2 = _tc_mid(acc1, y1, b1r, W2)
    acc2 = _sc_scatter(y2, ev)
    y3 = _tc_mid(acc2, y2, b2r, W3)
    acc3 = _sc_scatter(y3, ev)
    return _tc_fin(acc3, y3, b3r).T
